# baseline (speedup 1.0000x reference)
import jax
import jax.numpy as jnp
from jax.experimental import pallas as pl
from jax.experimental.pallas import tpu as pltpu

_BLOCK_I = 1024


def _spmm_kernel(xt_ref, w_ref, o_ref):
    i = pl.program_id(0)
    part = jax.lax.dot_general(
        xt_ref[:, pl.ds(i * _BLOCK_I, _BLOCK_I)], w_ref[...],
        dimension_numbers=(((1,), (0,)), ((), ())),
        preferred_element_type=jnp.float32,
    )

    @pl.when(i == 0)
    def _():
        o_ref[...] = part

    @pl.when(i > 0)
    def _():
        o_ref[...] += part


def kernel(input, W):
    size_in, cols = input.shape
    size_out = W.shape[1]
    xt = input.T
    out_t = pl.pallas_call(
        _spmm_kernel,
        grid=(size_in // _BLOCK_I,),
        in_specs=[
            pl.BlockSpec((cols, size_in), lambda i: (0, 0)),
            pl.BlockSpec((_BLOCK_I, size_out), lambda i: (i, 0)),
        ],
        out_specs=pl.BlockSpec((cols, size_out), lambda i: (0, 0)),
        out_shape=jax.ShapeDtypeStruct((cols, size_out), jnp.float32),
    )(xt, W)
    return out_t.T


# speedup vs baseline: 1.0765x; 1.0765x over previous
import jax
import jax.numpy as jnp
from jax.experimental import pallas as pl
from jax.experimental.pallas import tpu as pltpu

_BLOCK_I = 512


def _spmm_kernel(xt_ref, w_ref, o_ref):
    i = pl.program_id(0)
    part = jax.lax.dot_general(
        xt_ref[:, pl.ds(i * _BLOCK_I, _BLOCK_I)], w_ref[...],
        dimension_numbers=(((1,), (0,)), ((), ())),
        preferred_element_type=jnp.float32,
    )

    @pl.when(i == 0)
    def _():
        o_ref[...] = part

    @pl.when(i > 0)
    def _():
        o_ref[...] += part


def kernel(input, W):
    size_in, cols = input.shape
    size_out = W.shape[1]
    xt = input.T
    out_t = pl.pallas_call(
        _spmm_kernel,
        grid=(size_in // _BLOCK_I,),
        in_specs=[
            pl.BlockSpec((cols, size_in), lambda i: (0, 0)),
            pl.BlockSpec((_BLOCK_I, size_out), lambda i: (i, 0)),
        ],
        out_specs=pl.BlockSpec((cols, size_out), lambda i: (0, 0)),
        out_shape=jax.ShapeDtypeStruct((cols, size_out), jnp.float32),
    )(xt, W)
    return out_t.T
